# baseline (speedup 1.0000x reference)
import jax
import jax.numpy as jnp
from jax.experimental import pallas as pl
from jax.experimental.pallas import tpu as pltpu

_C = 80
_R = 2048
_B = 256
_SG = 128
_CC = 8
_NSG = _B // _SG
_NCC = _C // _CC


def _ce_kernel(lab_ref, x_ref, w_ref, loss_ref, acc_ref, corr_ref, num_ref):
    sg = pl.program_id(0)
    cc = pl.program_id(1)

    @pl.when(jnp.logical_and(sg == 0, cc == 0))
    def _init():
        loss_ref[0, 0] = 0.0
        corr_ref[0] = 0.0
        num_ref[0] = 0.0

    w = w_ref[...]
    lab = lab_ref[...]
    maskf_all = (lab > 0).astype(jnp.float32)
    n = jnp.sum(maskf_all, axis=1, keepdims=True)
    inv = 1.0 / (jnp.maximum(n, 1.0) * _B)
    lane = jax.lax.broadcasted_iota(jnp.int32, (_SG, _C), 1)

    loss_acc = jnp.zeros((_SG, 1), jnp.float32)
    corr_acc = jnp.zeros((_SG, 1), jnp.float32)
    num_acc = jnp.zeros((_SG, 1), jnp.float32)
    for j in range(_CC):
        c = cc * _CC + j
        xc = x_ref[:, j * _R:(j + 1) * _R]
        lt = jax.lax.dot_general(
            xc, w, (((1,), (1,)), ((), ())),
            preferred_element_type=jnp.float32)

        m = jnp.max(lt, axis=1, keepdims=True)
        lse = jnp.log(jnp.sum(jnp.exp(lt - m), axis=1, keepdims=True)) + m
        is_c = lane == c
        diag = jnp.sum(jnp.where(is_c, lt, 0.0), axis=1, keepdims=True)
        idx = jnp.min(jnp.where(lt == m, lane, _C), axis=1, keepdims=True)

        maskc = jnp.sum(jnp.where(is_c, maskf_all, 0.0), axis=1,
                        keepdims=True)
        loss_acc += (lse - diag) * maskc * inv
        corr_acc += jnp.where(idx == c, maskc, 0.0)
        num_acc += maskc

    loss_ref[0, 0] += jnp.sum(loss_acc)
    corr_ref[0] += jnp.sum(corr_acc)
    num_ref[0] += jnp.sum(num_acc)

    @pl.when(jnp.logical_and(sg == _NSG - 1, cc == _NCC - 1))
    def _fin():
        acc_ref[0, 0] = corr_ref[0] / num_ref[0]


def kernel(x, label, W):
    loss, acc = pl.pallas_call(
        _ce_kernel,
        grid=(_NSG, _NCC),
        in_specs=[
            pl.BlockSpec((_SG, _C), lambda sg, cc: (sg, 0)),
            pl.BlockSpec((_SG, _CC * _R), lambda sg, cc: (sg, cc)),
            pl.BlockSpec((_C, _R), lambda sg, cc: (0, 0)),
        ],
        out_specs=[
            pl.BlockSpec(memory_space=pltpu.SMEM),
            pl.BlockSpec(memory_space=pltpu.SMEM),
        ],
        out_shape=[
            jax.ShapeDtypeStruct((1, 1), jnp.float32),
            jax.ShapeDtypeStruct((1, 1), jnp.float32),
        ],
        scratch_shapes=[
            pltpu.SMEM((1,), jnp.float32),
            pltpu.SMEM((1,), jnp.float32),
        ],
        compiler_params=pltpu.CompilerParams(
            dimension_semantics=("arbitrary", "arbitrary")),
    )(label, x, W)
    return loss.reshape(()), acc.reshape(())


# speedup vs baseline: 1.0533x; 1.0533x over previous
import jax
import jax.numpy as jnp
from jax.experimental import pallas as pl
from jax.experimental.pallas import tpu as pltpu

_C = 80
_R = 2048
_B = 256
_SG = 128
_CC = 16
_NSG = _B // _SG
_NCC = _C // _CC


def _ce_kernel(lab_ref, x_ref, w_ref, loss_ref, acc_ref, corr_ref, num_ref):
    sg = pl.program_id(0)
    cc = pl.program_id(1)

    @pl.when(jnp.logical_and(sg == 0, cc == 0))
    def _init():
        loss_ref[0, 0] = 0.0
        corr_ref[0] = 0.0
        num_ref[0] = 0.0

    w = w_ref[...]
    lab = lab_ref[...]
    maskf_all = (lab > 0).astype(jnp.float32)
    n = jnp.sum(maskf_all, axis=1, keepdims=True)
    inv = 1.0 / (jnp.maximum(n, 1.0) * _B)
    lane = jax.lax.broadcasted_iota(jnp.int32, (_SG, _C), 1)

    loss_acc = jnp.zeros((_SG, 1), jnp.float32)
    corr_acc = jnp.zeros((_SG, 1), jnp.float32)
    num_acc = jnp.zeros((_SG, 1), jnp.float32)
    for j in range(_CC):
        c = cc * _CC + j
        xc = x_ref[:, j * _R:(j + 1) * _R]
        lt = jax.lax.dot_general(
            xc, w, (((1,), (1,)), ((), ())),
            preferred_element_type=jnp.float32)

        m = jnp.max(lt, axis=1, keepdims=True)
        lse = jnp.log(jnp.sum(jnp.exp(lt - m), axis=1, keepdims=True)) + m
        is_c = lane == c
        diag = jnp.sum(jnp.where(is_c, lt, 0.0), axis=1, keepdims=True)
        idx = jnp.min(jnp.where(lt == m, lane, _C), axis=1, keepdims=True)

        maskc = jnp.sum(jnp.where(is_c, maskf_all, 0.0), axis=1,
                        keepdims=True)
        loss_acc += (lse - diag) * maskc * inv
        corr_acc += jnp.where(idx == c, maskc, 0.0)
        num_acc += maskc

    loss_ref[0, 0] += jnp.sum(loss_acc)
    corr_ref[0] += jnp.sum(corr_acc)
    num_ref[0] += jnp.sum(num_acc)

    @pl.when(jnp.logical_and(sg == _NSG - 1, cc == _NCC - 1))
    def _fin():
        acc_ref[0, 0] = corr_ref[0] / num_ref[0]


def kernel(x, label, W):
    loss, acc = pl.pallas_call(
        _ce_kernel,
        grid=(_NSG, _NCC),
        in_specs=[
            pl.BlockSpec((_SG, _C), lambda sg, cc: (sg, 0)),
            pl.BlockSpec((_SG, _CC * _R), lambda sg, cc: (sg, cc)),
            pl.BlockSpec((_C, _R), lambda sg, cc: (0, 0)),
        ],
        out_specs=[
            pl.BlockSpec(memory_space=pltpu.SMEM),
            pl.BlockSpec(memory_space=pltpu.SMEM),
        ],
        out_shape=[
            jax.ShapeDtypeStruct((1, 1), jnp.float32),
            jax.ShapeDtypeStruct((1, 1), jnp.float32),
        ],
        scratch_shapes=[
            pltpu.SMEM((1,), jnp.float32),
            pltpu.SMEM((1,), jnp.float32),
        ],
        compiler_params=pltpu.CompilerParams(
            dimension_semantics=("arbitrary", "arbitrary")),
    )(label, x, W)
    return loss.reshape(()), acc.reshape(())
